# baseline (speedup 1.0000x reference)
import functools

import jax
import jax.numpy as jnp
from jax import lax
from jax.experimental import pallas as pl
from jax.experimental.pallas import tpu as pltpu
import jax.experimental.pallas.tpu_sc as plsc

TOTAL_TOKENS = 16384
HIDDEN = 256
N_SPANS = 32768
NW = 32
SPW = N_SPANS // NW
CHUNK = 64
NCHUNK = SPW // CHUNK
IDXR = N_SPANS // CHUNK
PBLK = 1024
MBLK = 4096
NSEG = 16
BIG = 0x7FFFFFFF


def _prefix_body(x_ref, starts_ref, lens_ref, cu_ref,
                 t_ref, ends_ref, inv_ref, carry_ref):
    b = pl.program_id(0)

    @pl.when(b == 0)
    def _():
        carry_ref[...] = jnp.zeros_like(carry_ref)

    s = starts_ref[...]
    l = lens_ref[...] + 1
    se = jnp.full(s.shape, BIG, jnp.int32)
    for k in range(1, NSEG + 1):
        cb = cu_ref[0, k]
        se = jnp.minimum(se, jnp.where(cb > s, cb, BIG))
    e = jnp.minimum(s + l, se)
    eff = jnp.maximum(e - s, 1)
    ends_ref[...] = jnp.concatenate([e, s], axis=1)
    inv_ref[...] = 1.0 / eff.astype(jnp.float32)

    x = x_ref[...]
    ii = lax.broadcasted_iota(jnp.int32, (PBLK, PBLK), 0)
    jj = lax.broadcasted_iota(jnp.int32, (PBLK, PBLK), 1)
    ltri = (jj < ii).astype(jnp.float32)
    t_ref[...] = (
        jnp.dot(ltri, x, preferred_element_type=jnp.float32) + carry_ref[...]
    )
    carry_ref[...] = carry_ref[...] + jnp.sum(x, axis=0, keepdims=True)


def _prefix_sums(tokens, span_starts, span_lengths, cu_pad):
    nblk = TOTAL_TOKENS // PBLK
    starts2 = jnp.reshape(span_starts, (IDXR, CHUNK))
    lens2 = jnp.reshape(span_lengths, (IDXR, CHUNK))
    return pl.pallas_call(
        _prefix_body,
        grid=(nblk,),
        in_specs=[
            pl.BlockSpec((PBLK, HIDDEN), lambda b: (b, 0)),
            pl.BlockSpec((IDXR // (TOTAL_TOKENS // PBLK), CHUNK),
                         lambda b: (b, 0)),
            pl.BlockSpec((IDXR // (TOTAL_TOKENS // PBLK), CHUNK),
                         lambda b: (b, 0)),
            pl.BlockSpec(memory_space=pltpu.SMEM),
        ],
        out_specs=[
            pl.BlockSpec((PBLK, HIDDEN), lambda b: (b, 0)),
            pl.BlockSpec((IDXR // (TOTAL_TOKENS // PBLK), 2 * CHUNK),
                         lambda b: (b, 0)),
            pl.BlockSpec((IDXR // (TOTAL_TOKENS // PBLK), CHUNK),
                         lambda b: (b, 0)),
        ],
        out_shape=[
            jax.ShapeDtypeStruct((TOTAL_TOKENS, HIDDEN), jnp.float32),
            jax.ShapeDtypeStruct((IDXR, 2 * CHUNK), jnp.int32),
            jax.ShapeDtypeStruct((IDXR, CHUNK), jnp.float32),
        ],
        scratch_shapes=[pltpu.VMEM((1, HIDDEN), jnp.float32)],
    )(tokens, starts2, lens2, cu_pad)


def _sc_body(t_hbm, idx_hbm, inv_hbm, out_hbm,
             idx_v, inv_v, buf0, buf1, buf2, inv_smem,
             sem_g0, sem_g1, sem_g2, sem_w0, sem_w1, sem_w2):
    wid = lax.axis_index("s") * 2 + lax.axis_index("c")
    rbase = wid * NCHUNK
    pltpu.sync_copy(idx_hbm.at[pl.ds(rbase, NCHUNK)], idx_v)
    pltpu.sync_copy(inv_hbm.at[pl.ds(rbase, NCHUNK)], inv_v)

    bufs = (buf0, buf1, buf2)
    gsems = (sem_g0, sem_g1, sem_g2)
    wsems = (sem_w0, sem_w1, sem_w2)

    def gather(c, slot):
        return pltpu.async_copy(t_hbm.at[idx_v.at[c]], bufs[slot],
                                gsems[slot])

    def wr(c, slot):
        return pltpu.async_copy(
            bufs[slot].at[pl.ds(0, CHUNK)],
            out_hbm.at[pl.ds(wid * SPW + c * CHUNK, CHUNK)],
            wsems[slot])

    def scale(c, slot):
        buf = bufs[slot]

        def body(g, _):
            r0 = g * 2
            r1 = g * 2 + 1
            iv0 = inv_smem[c * CHUNK + r0]
            iv1 = inv_smem[c * CHUNK + r1]
            for j in range(HIDDEN // 16):
                js = pl.ds(j * 16, 16)
                buf[r0, js] = (buf[r0, js] - buf[CHUNK + r0, js]) * iv0
            for j in range(HIDDEN // 16):
                js = pl.ds(j * 16, 16)
                buf[r1, js] = (buf[r1, js] - buf[CHUNK + r1, js]) * iv1
            return 0

        lax.fori_loop(0, CHUNK // 2, body, 0)

    pg = {}
    pw = {}
    pg[0] = gather(0, 0)
    pg[1] = gather(1, 1)

    def fill_body(r, _):
        for g in range(CHUNK // 16):
            iv16 = inv_v[r, pl.ds(g * 16, 16)]
            for k in range(16):
                inv_smem[r * CHUNK + g * 16 + k] = iv16[k]
        return 0

    lax.fori_loop(0, NCHUNK, fill_body, 0)

    for c in range(NCHUNK):
        slot = c % 3
        pg[c].wait()
        scale(c, slot)
        pw[c] = wr(c, slot)
        if c + 2 < NCHUNK:
            if c >= 1:
                pw[c - 1].wait()
            pg[c + 2] = gather(c + 2, (c + 2) % 3)
    pw[NCHUNK - 3].wait()
    pw[NCHUNK - 2].wait()
    pw[NCHUNK - 1].wait()


def _sc_span_embs(t, idx_comb, inv):
    mesh = plsc.VectorSubcoreMesh(core_axis_name="c", subcore_axis_name="s")
    fn = functools.partial(
        pl.kernel,
        out_type=jax.ShapeDtypeStruct((N_SPANS, HIDDEN), jnp.float32),
        mesh=mesh,
        scratch_types=[
            pltpu.VMEM((NCHUNK, 2 * CHUNK), jnp.int32),
            pltpu.VMEM((NCHUNK, CHUNK), jnp.float32),
            pltpu.VMEM((2 * CHUNK, HIDDEN), jnp.float32),
            pltpu.VMEM((2 * CHUNK, HIDDEN), jnp.float32),
            pltpu.VMEM((2 * CHUNK, HIDDEN), jnp.float32),
            pltpu.SMEM((SPW,), jnp.float32),
            pltpu.SemaphoreType.DMA,
            pltpu.SemaphoreType.DMA,
            pltpu.SemaphoreType.DMA,
            pltpu.SemaphoreType.DMA,
            pltpu.SemaphoreType.DMA,
            pltpu.SemaphoreType.DMA,
        ],
    )(_sc_body)
    return fn(t, idx_comb, inv)


def _mlp_body(x_ref, w1_ref, b1_ref, w2_ref, b2_ref, y_ref,
              scores_ref, loss_ref):
    b = pl.program_id(0)
    x = x_ref[...]
    h = jnp.maximum(
        jnp.dot(x.astype(jnp.bfloat16), w1_ref[...].astype(jnp.bfloat16),
                preferred_element_type=jnp.float32)
        + b1_ref[...],
        0.0,
    )
    lcol = jnp.dot(h.astype(jnp.bfloat16), w2_ref[...].astype(jnp.bfloat16),
                   preferred_element_type=jnp.float32) + b2_ref[0, 0]
    logits2 = jnp.reshape(lcol, (MBLK // 128, 128))
    ea = jnp.exp(-jnp.abs(logits2))
    scores_ref[...] = jnp.where(logits2 >= 0.0, 1.0 / (1.0 + ea),
                                ea / (1.0 + ea))
    y = y_ref[...].astype(jnp.float32)
    part = (
        jnp.maximum(logits2, 0.0)
        - logits2 * y
        + jnp.log1p(ea)
    )

    @pl.when(b == 0)
    def _():
        loss_ref[0, 0] = 0.0

    loss_ref[0, 0] += jnp.sum(part)

    @pl.when(b == (N_SPANS // MBLK) - 1)
    def _():
        loss_ref[0, 0] = loss_ref[0, 0] * (1.0 / N_SPANS)


def _mlp(span_embs, W1, b1, W2, b2, labels):
    ng = N_SPANS // MBLK
    scores2, loss2 = pl.pallas_call(
        _mlp_body,
        grid=(ng,),
        in_specs=[
            pl.BlockSpec((MBLK, HIDDEN), lambda b: (b, 0)),
            pl.BlockSpec((HIDDEN, HIDDEN), lambda b: (0, 0)),
            pl.BlockSpec((1, HIDDEN), lambda b: (0, 0)),
            pl.BlockSpec((HIDDEN, 1), lambda b: (0, 0)),
            pl.BlockSpec(memory_space=pltpu.SMEM),
            pl.BlockSpec((MBLK // 128, 128), lambda b: (b, 0)),
        ],
        out_specs=[
            pl.BlockSpec((MBLK // 128, 128), lambda b: (b, 0)),
            pl.BlockSpec(memory_space=pltpu.SMEM),
        ],
        out_shape=[
            jax.ShapeDtypeStruct((N_SPANS // 128, 128), jnp.float32),
            jax.ShapeDtypeStruct((1, 1), jnp.float32),
        ],
    )(span_embs, W1, jnp.reshape(b1, (1, HIDDEN)), W2,
      jnp.reshape(b2, (1, 1)), jnp.reshape(labels, (N_SPANS // 128, 128)))
    return jnp.reshape(scores2, (N_SPANS,)), jnp.reshape(loss2, ())


def kernel(tokens, W1, b1, W2, b2, cu_seqlens, span_starts, span_lengths,
           span_labels):
    cu_pad = jnp.reshape(
        jnp.concatenate([cu_seqlens, jnp.full((15,), BIG, jnp.int32)]),
        (1, 32))
    t, idx_comb, inv = _prefix_sums(tokens, span_starts, span_lengths,
                                    cu_pad)
    span_embs = _sc_span_embs(t, idx_comb, inv)
    scores, loss = _mlp(span_embs, W1, b1, W2, b2, span_labels)
    return span_embs, scores, loss


# speedup vs baseline: 1.1741x; 1.1741x over previous
import functools

import jax
import jax.numpy as jnp
from jax import lax
from jax.experimental import pallas as pl
from jax.experimental.pallas import tpu as pltpu
import jax.experimental.pallas.tpu_sc as plsc

TOTAL_TOKENS = 16384
HIDDEN = 256
N_SPANS = 32768
NW = 32
SPW = N_SPANS // NW
CHUNK = 64
NCHUNK = SPW // CHUNK
IDXR = N_SPANS // CHUNK
PBLK = 1024
MBLK = 4096
NSEG = 16
BIG = 0x7FFFFFFF


def _prefix_body(x_ref, starts_ref, lens_ref, cu_ref,
                 t_ref, ends_ref, inv_ref, carry_ref):
    b = pl.program_id(0)

    @pl.when(b == 0)
    def _():
        carry_ref[...] = jnp.zeros_like(carry_ref)

    s = starts_ref[...]
    l = lens_ref[...] + 1
    se = jnp.full(s.shape, BIG, jnp.int32)
    for k in range(1, NSEG + 1):
        cb = cu_ref[0, k]
        se = jnp.minimum(se, jnp.where(cb > s, cb, BIG))
    e = jnp.minimum(s + l, se)
    eff = jnp.maximum(e - s, 1)
    ends_ref[...] = e
    inv_ref[...] = 1.0 / eff.astype(jnp.float32)

    x = x_ref[...]
    ii = lax.broadcasted_iota(jnp.int32, (PBLK, PBLK), 0)
    jj = lax.broadcasted_iota(jnp.int32, (PBLK, PBLK), 1)
    ltri = (jj < ii).astype(jnp.float32)
    t_ref[...] = (
        jnp.dot(ltri, x, preferred_element_type=jnp.float32) + carry_ref[...]
    )
    carry_ref[...] = carry_ref[...] + jnp.sum(x, axis=0, keepdims=True)


def _prefix_sums(tokens, span_starts, span_lengths, cu_pad):
    nblk = TOTAL_TOKENS // PBLK
    starts2 = jnp.reshape(span_starts, (IDXR, CHUNK))
    lens2 = jnp.reshape(span_lengths, (IDXR, CHUNK))
    return pl.pallas_call(
        _prefix_body,
        grid=(nblk,),
        in_specs=[
            pl.BlockSpec((PBLK, HIDDEN), lambda b: (b, 0)),
            pl.BlockSpec((IDXR // (TOTAL_TOKENS // PBLK), CHUNK),
                         lambda b: (b, 0)),
            pl.BlockSpec((IDXR // (TOTAL_TOKENS // PBLK), CHUNK),
                         lambda b: (b, 0)),
            pl.BlockSpec(memory_space=pltpu.SMEM),
        ],
        out_specs=[
            pl.BlockSpec((PBLK, HIDDEN), lambda b: (b, 0)),
            pl.BlockSpec((IDXR // (TOTAL_TOKENS // PBLK), CHUNK),
                         lambda b: (b, 0)),
            pl.BlockSpec((IDXR // (TOTAL_TOKENS // PBLK), CHUNK),
                         lambda b: (b, 0)),
        ],
        out_shape=[
            jax.ShapeDtypeStruct((TOTAL_TOKENS, HIDDEN), jnp.float32),
            jax.ShapeDtypeStruct((IDXR, CHUNK), jnp.int32),
            jax.ShapeDtypeStruct((IDXR, CHUNK), jnp.float32),
        ],
        scratch_shapes=[pltpu.VMEM((1, HIDDEN), jnp.float32)],
    )(tokens, starts2, lens2, cu_pad)


def _sc_body(t_hbm, idxe_hbm, idxs_hbm, inv_hbm, out_hbm,
             idxe_v, idxs_v, inv_v,
             buf_e0, buf_e1, buf_e2, buf_s0, buf_s1, buf_s2, inv_smem,
             sem_g0, sem_g1, sem_g2, sem_w0, sem_w1, sem_w2):
    wid = lax.axis_index("s") * 2 + lax.axis_index("c")
    rbase = wid * NCHUNK
    pltpu.sync_copy(idxe_hbm.at[pl.ds(rbase, NCHUNK)], idxe_v)
    pltpu.sync_copy(idxs_hbm.at[pl.ds(rbase, NCHUNK)], idxs_v)
    pltpu.sync_copy(inv_hbm.at[pl.ds(rbase, NCHUNK)], inv_v)

    ebufs = (buf_e0, buf_e1, buf_e2)
    sbufs = (buf_s0, buf_s1, buf_s2)
    gsems = (sem_g0, sem_g1, sem_g2)
    wsems = (sem_w0, sem_w1, sem_w2)

    def gather(c, slot):
        return (
            pltpu.async_copy(t_hbm.at[idxe_v.at[c]], ebufs[slot],
                             gsems[slot]),
            pltpu.async_copy(t_hbm.at[idxs_v.at[c]], sbufs[slot],
                             gsems[slot]),
        )

    def wr(c, slot):
        return pltpu.async_copy(
            ebufs[slot], out_hbm.at[pl.ds(wid * SPW + c * CHUNK, CHUNK)],
            wsems[slot])

    def scale(c, slot):
        be = ebufs[slot]
        bs = sbufs[slot]

        def body(g, _):
            r0 = g * 2
            r1 = g * 2 + 1
            iv0 = inv_smem[c * CHUNK + r0]
            iv1 = inv_smem[c * CHUNK + r1]
            for j in range(HIDDEN // 16):
                js = pl.ds(j * 16, 16)
                be[r0, js] = (be[r0, js] - bs[r0, js]) * iv0
            for j in range(HIDDEN // 16):
                js = pl.ds(j * 16, 16)
                be[r1, js] = (be[r1, js] - bs[r1, js]) * iv1
            return 0

        lax.fori_loop(0, CHUNK // 2, body, 0)

    pg = {}
    pw = {}
    pg[0] = gather(0, 0)
    pg[1] = gather(1, 1)

    def fill_body(r, _):
        for g in range(CHUNK // 16):
            iv16 = inv_v[r, pl.ds(g * 16, 16)]
            for k in range(16):
                inv_smem[r * CHUNK + g * 16 + k] = iv16[k]
        return 0

    lax.fori_loop(0, NCHUNK, fill_body, 0)

    for c in range(NCHUNK):
        slot = c % 3
        pg[c][0].wait()
        pg[c][1].wait()
        scale(c, slot)
        pw[c] = wr(c, slot)
        if c + 2 < NCHUNK:
            if c >= 1:
                pw[c - 1].wait()
            pg[c + 2] = gather(c + 2, (c + 2) % 3)
    pw[NCHUNK - 3].wait()
    pw[NCHUNK - 2].wait()
    pw[NCHUNK - 1].wait()


def _sc_span_embs(t, idx_e, idx_s, inv):
    mesh = plsc.VectorSubcoreMesh(core_axis_name="c", subcore_axis_name="s")
    fn = functools.partial(
        pl.kernel,
        out_type=jax.ShapeDtypeStruct((N_SPANS, HIDDEN), jnp.float32),
        mesh=mesh,
        scratch_types=[
            pltpu.VMEM((NCHUNK, CHUNK), jnp.int32),
            pltpu.VMEM((NCHUNK, CHUNK), jnp.int32),
            pltpu.VMEM((NCHUNK, CHUNK), jnp.float32),
            pltpu.VMEM((CHUNK, HIDDEN), jnp.float32),
            pltpu.VMEM((CHUNK, HIDDEN), jnp.float32),
            pltpu.VMEM((CHUNK, HIDDEN), jnp.float32),
            pltpu.VMEM((CHUNK, HIDDEN), jnp.float32),
            pltpu.VMEM((CHUNK, HIDDEN), jnp.float32),
            pltpu.VMEM((CHUNK, HIDDEN), jnp.float32),
            pltpu.SMEM((SPW,), jnp.float32),
            pltpu.SemaphoreType.DMA,
            pltpu.SemaphoreType.DMA,
            pltpu.SemaphoreType.DMA,
            pltpu.SemaphoreType.DMA,
            pltpu.SemaphoreType.DMA,
            pltpu.SemaphoreType.DMA,
        ],
    )(_sc_body)
    return fn(t, idx_e, idx_s, inv)


def _mlp_body(x_ref, w1_ref, b1_ref, w2_ref, b2_ref, y_ref,
              scores_ref, loss_ref):
    b = pl.program_id(0)
    x = x_ref[...]
    h = jnp.maximum(
        jnp.dot(x.astype(jnp.bfloat16), w1_ref[...].astype(jnp.bfloat16),
                preferred_element_type=jnp.float32)
        + b1_ref[...],
        0.0,
    )
    lcol = jnp.dot(h.astype(jnp.bfloat16), w2_ref[...].astype(jnp.bfloat16),
                   preferred_element_type=jnp.float32) + b2_ref[0, 0]
    logits2 = jnp.reshape(lcol, (MBLK // 128, 128))
    ea = jnp.exp(-jnp.abs(logits2))
    scores_ref[...] = jnp.where(logits2 >= 0.0, 1.0 / (1.0 + ea),
                                ea / (1.0 + ea))
    y = y_ref[...].astype(jnp.float32)
    part = (
        jnp.maximum(logits2, 0.0)
        - logits2 * y
        + jnp.log1p(ea)
    )

    @pl.when(b == 0)
    def _():
        loss_ref[0, 0] = 0.0

    loss_ref[0, 0] += jnp.sum(part)

    @pl.when(b == (N_SPANS // MBLK) - 1)
    def _():
        loss_ref[0, 0] = loss_ref[0, 0] * (1.0 / N_SPANS)


def _mlp(span_embs, W1, b1, W2, b2, labels):
    ng = N_SPANS // MBLK
    scores2, loss2 = pl.pallas_call(
        _mlp_body,
        grid=(ng,),
        in_specs=[
            pl.BlockSpec((MBLK, HIDDEN), lambda b: (b, 0)),
            pl.BlockSpec((HIDDEN, HIDDEN), lambda b: (0, 0)),
            pl.BlockSpec((1, HIDDEN), lambda b: (0, 0)),
            pl.BlockSpec((HIDDEN, 1), lambda b: (0, 0)),
            pl.BlockSpec(memory_space=pltpu.SMEM),
            pl.BlockSpec((MBLK // 128, 128), lambda b: (b, 0)),
        ],
        out_specs=[
            pl.BlockSpec((MBLK // 128, 128), lambda b: (b, 0)),
            pl.BlockSpec(memory_space=pltpu.SMEM),
        ],
        out_shape=[
            jax.ShapeDtypeStruct((N_SPANS // 128, 128), jnp.float32),
            jax.ShapeDtypeStruct((1, 1), jnp.float32),
        ],
    )(span_embs, W1, jnp.reshape(b1, (1, HIDDEN)), W2,
      jnp.reshape(b2, (1, 1)), jnp.reshape(labels, (N_SPANS // 128, 128)))
    return jnp.reshape(scores2, (N_SPANS,)), jnp.reshape(loss2, ())


def kernel(tokens, W1, b1, W2, b2, cu_seqlens, span_starts, span_lengths,
           span_labels):
    cu_pad = jnp.reshape(
        jnp.concatenate([cu_seqlens, jnp.full((15,), BIG, jnp.int32)]),
        (1, 32))
    t, idx_e, inv = _prefix_sums(tokens, span_starts, span_lengths, cu_pad)
    idx_s = jnp.reshape(span_starts, (IDXR, CHUNK))
    span_embs = _sc_span_embs(t, idx_e, idx_s, inv)
    scores, loss = _mlp(span_embs, W1, b1, W2, b2, span_labels)
    return span_embs, scores, loss
